# baseline (speedup 1.0000x reference)
import functools

import jax
import jax.numpy as jnp
from jax import lax
from jax.experimental import pallas as pl
from jax.experimental.pallas import tpu as pltpu
from jax.experimental.pallas import tpu_sc as plsc

_VOCAB = 100000
_EMB_D = 200
_B, _L = 16, 512
_N = 10000
_E = 320000
_F = 128
_C = 384

_NC, _NS = 2, 16
_NW = _NC * _NS
_CHUNK = 128
_K_SEG = -(-_E // (_NW * _CHUNK))
_E_PAD = _NW * _K_SEG * _CHUNK
_ACC_ROWS = 10240
_ZROWS = _ACC_ROWS // _NS

_IDS = _B * _L
_K_EMB = _IDS // (_NW * _CHUNK)


@functools.partial(
    pl.kernel,
    out_type=[jax.ShapeDtypeStruct((_IDS, _F), jnp.float32),
              jax.ShapeDtypeStruct((_IDS, _F), jnp.float32)],
    mesh=plsc.VectorSubcoreMesh(core_axis_name="c", subcore_axis_name="s"),
    scratch_types=[
        pltpu.VMEM((_K_EMB, _CHUNK), jnp.int32),
        pltpu.VMEM((_CHUNK, _F), jnp.float32),
        pltpu.VMEM((_CHUNK, _F), jnp.float32),
        pltpu.SemaphoreType.DMA,
    ],
)
def _sc_embed(emb_hbm, hi_hbm, ids_hbm, lo_out, hi_out, ids_v, rlo, rhi, sem):
    wid = lax.axis_index("c") * _NS + lax.axis_index("s")
    pltpu.sync_copy(ids_hbm.at[wid], ids_v)
    for j in range(_K_EMB):
        base = wid * (_K_EMB * _CHUNK) + j * _CHUNK
        pltpu.async_copy(
            emb_hbm.at[ids_v.at[j], pl.ds(0, _F)], rlo, sem).wait()
        pltpu.sync_copy(rlo, lo_out.at[pl.ds(base, _CHUNK)])
        pltpu.async_copy(hi_hbm.at[ids_v.at[j]], rhi, sem).wait()
        pltpu.sync_copy(rhi, hi_out.at[pl.ds(base, _CHUNK)])


@functools.partial(
    pl.kernel,
    out_type=jax.ShapeDtypeStruct((_NC, _ACC_ROWS, _F), jnp.float32),
    mesh=plsc.VectorSubcoreMesh(core_axis_name="c", subcore_axis_name="s"),
    scratch_types=[
        pltpu.VMEM((_K0, _CHUNK), jnp.int32),
        pltpu.VMEM((_K0, _CHUNK), jnp.int32),
        pltpu.VMEM((_CHUNK, _F), jnp.float32),
        pltpu.VMEM_SHARED((_ACC_ROWS, _F), jnp.float32),
        pltpu.SemaphoreType.DMA,
    ],
)
def _sc_segsum(x_hbm, src_hbm, dst_hbm, zeros_hbm, out_hbm,
               src_v, dst_v, rows_v, acc_sh, sem):
    cid = lax.axis_index("c")
    sid = lax.axis_index("s")
    base = (1 - cid) * sid * _K0 + cid * (_NS * _K0 + sid * _K1)
    pltpu.sync_copy(src_hbm.at[pl.ds(base, _K0)], src_v)
    pltpu.sync_copy(dst_hbm.at[pl.ds(base, _K0)], dst_v)
    pltpu.sync_copy(zeros_hbm, acc_sh.at[pl.ds(sid * _ZROWS, _ZROWS)])
    plsc.subcore_barrier()

    def body(j, carry):
        pltpu.async_copy(x_hbm.at[src_v.at[j]], rows_v, sem).wait()
        pltpu.sync_copy(rows_v, acc_sh.at[dst_v.at[j]], add=True)
        return carry

    @pl.when(cid == 0)
    def _():
        lax.fori_loop(0, _K0, body, 0)

    @pl.when(cid == 1)
    def _():
        lax.fori_loop(0, _K1, body, 0)

    plsc.subcore_barrier()
    pltpu.sync_copy(acc_sh.at[pl.ds(sid * _ZROWS, _ZROWS)],
                    out_hbm.at[cid, pl.ds(sid * _ZROWS, _ZROWS)])


def _tc_conv_body(elo_ref, ehi_ref, wlo_ref, whi_ref, b3_ref, b4_ref, b5_ref,
                  out_ref):
    y = (jnp.dot(elo_ref[0], wlo_ref[...], preferred_element_type=jnp.float32)
         + jnp.dot(ehi_ref[0], whi_ref[...], preferred_element_type=jnp.float32))
    c3 = y[0:510, 0:128] + y[1:511, 128:256] + y[2:512, 256:384]
    m3 = jnp.max(c3, axis=0, keepdims=True) + b3_ref[...]
    c4 = (y[0:509, 384:512] + y[1:510, 512:640]
          + y[2:511, 640:768] + y[3:512, 768:896])
    m4 = jnp.max(c4, axis=0, keepdims=True) + b4_ref[...]
    c5 = (y[0:508, 896:1024] + y[1:509, 1024:1152] + y[2:510, 1152:1280]
          + y[3:511, 1280:1408] + y[4:512, 1408:1536])
    m5 = jnp.max(c5, axis=0, keepdims=True) + b5_ref[...]
    row = jnp.maximum(jnp.concatenate([m3, m4, m5], axis=1), 0.0)
    mx = jnp.max(row, axis=1, keepdims=True)
    ls = row - mx - jnp.log(jnp.sum(jnp.exp(row - mx), axis=1, keepdims=True))
    out_ref[pl.ds(pl.program_id(0), 1), :] = ls


def _tc_conv(e_lo, e_hi, wlo, whi, b3, b4, b5):
    return pl.pallas_call(
        _tc_conv_body,
        grid=(_B,),
        in_specs=[
            pl.BlockSpec((1, _L, _F), lambda b: (b, 0, 0)),
            pl.BlockSpec((1, _L, _F), lambda b: (b, 0, 0)),
            pl.BlockSpec((_F, 1536), lambda b: (0, 0)),
            pl.BlockSpec((_F, 1536), lambda b: (0, 0)),
            pl.BlockSpec((1, 128), lambda b: (0, 0)),
            pl.BlockSpec((1, 128), lambda b: (0, 0)),
            pl.BlockSpec((1, 128), lambda b: (0, 0)),
        ],
        out_specs=pl.BlockSpec((_B, _C), lambda b: (0, 0)),
        out_shape=jax.ShapeDtypeStruct((_B, _C), jnp.float32),
    )(e_lo, e_hi, wlo, whi, b3, b4, b5)


_SPLIT_BLK = 2000


def _tc_split_body(x_any, hi_ref, buf, sem):
    i = pl.program_id(0)
    pltpu.make_async_copy(
        x_any.at[pl.ds(i * _SPLIT_BLK, _SPLIT_BLK), :], buf, sem).start()
    pltpu.make_async_copy(
        x_any.at[pl.ds(i * _SPLIT_BLK, _SPLIT_BLK), :], buf, sem).wait()
    hi_ref[...] = jnp.concatenate(
        [buf[:, _F:_EMB_D],
         jnp.zeros((_SPLIT_BLK, 2 * _F - _EMB_D), jnp.float32)], axis=1)


def _tc_split_emb(emb):
    return pl.pallas_call(
        _tc_split_body,
        grid=(_VOCAB // _SPLIT_BLK,),
        in_specs=[pl.BlockSpec(memory_space=pltpu.MemorySpace.HBM)],
        out_specs=pl.BlockSpec((_SPLIT_BLK, _F), lambda i: (i, 0)),
        out_shape=jax.ShapeDtypeStruct((_VOCAB, _F), jnp.float32),
        scratch_shapes=[pltpu.VMEM((_SPLIT_BLK, _EMB_D), jnp.float32),
                        pltpu.SemaphoreType.DMA],
    )(emb)


def _tc_matmul_body(x_ref, w_ref, o_ref):
    o_ref[...] = jnp.dot(x_ref[...], w_ref[...],
                         preferred_element_type=jnp.float32)


def _tc_matmul(x, w, rows_blk):
    n, d = x.shape
    return pl.pallas_call(
        _tc_matmul_body,
        grid=(n // rows_blk,),
        in_specs=[
            pl.BlockSpec((rows_blk, d), lambda i: (i, 0)),
            pl.BlockSpec(w.shape, lambda i: (0, 0)),
        ],
        out_specs=pl.BlockSpec((rows_blk, w.shape[1]), lambda i: (i, 0)),
        out_shape=jax.ShapeDtypeStruct((n, w.shape[1]), jnp.float32),
    )(x, w)


def _tc_relu_body(a_ref, b_ref, bias_ref, o_ref):
    o_ref[...] = jnp.maximum(a_ref[0] + b_ref[0] + bias_ref[...], 0.0)


def _tc_relu_combine(p, bias, rows_blk, n=_N):
    d = p.shape[2]
    return pl.pallas_call(
        _tc_relu_body,
        grid=(n // rows_blk,),
        in_specs=[
            pl.BlockSpec((1, rows_blk, d), lambda i: (0, i, 0)),
            pl.BlockSpec((1, rows_blk, d), lambda i: (1, i, 0)),
            pl.BlockSpec((1, d), lambda i: (0, 0)),
        ],
        out_specs=pl.BlockSpec((rows_blk, d), lambda i: (i, 0)),
        out_shape=jax.ShapeDtypeStruct((n, d), jnp.float32),
    )(p, p, bias)


def _tc_final_body(a_ref, b_ref, w_ref, bias_ref, o_ref):
    s = a_ref[0] + b_ref[0]
    y = jnp.dot(s, w_ref[...], preferred_element_type=jnp.float32) + bias_ref[...]
    mx = jnp.max(y, axis=1, keepdims=True)
    o_ref[...] = y - mx - jnp.log(
        jnp.sum(jnp.exp(y - mx), axis=1, keepdims=True))


def _tc_final(p, w, bias, rows_blk, n=_N):
    d = p.shape[2]
    return pl.pallas_call(
        _tc_final_body,
        grid=(n // rows_blk,),
        in_specs=[
            pl.BlockSpec((1, rows_blk, d), lambda i: (0, i, 0)),
            pl.BlockSpec((1, rows_blk, d), lambda i: (1, i, 0)),
            pl.BlockSpec(w.shape, lambda i: (0, 0)),
            pl.BlockSpec((1, w.shape[1]), lambda i: (0, 0)),
        ],
        out_specs=pl.BlockSpec((rows_blk, w.shape[1]), lambda i: (i, 0)),
        out_shape=jax.ShapeDtypeStruct((n, w.shape[1]), jnp.float32),
    )(p, p, w, bias)


def kernel(input_seq, edge_index, features, emb, W3, b3, W4, b4, W5, b5,
           Wg1, bg1, Wg2, bg2):
    f32 = jnp.float32
    ids = input_seq.astype(jnp.int32).reshape(_NW, _K_EMB, _CHUNK)
    src = edge_index[0].astype(jnp.int32)
    dst = edge_index[1].astype(jnp.int32)
    src_p = jnp.pad(src, (0, _E_PAD - _E)).reshape(_NW, _K_SEG, _CHUNK)
    dst_p = jnp.pad(dst, (0, _E_PAD - _E),
                    constant_values=_N).reshape(_NW, _K_SEG, _CHUNK)
    zeros = jnp.zeros((_ZROWS, _F), f32)
    w3t = jnp.transpose(W3[:, 0], (1, 2, 0))
    w4t = jnp.transpose(W4[:, 0], (1, 2, 0))
    w5t = jnp.transpose(W5[:, 0], (1, 2, 0))
    wcat = jnp.concatenate([w3t, w4t, w5t], axis=0)
    wcat = jnp.transpose(wcat, (1, 0, 2)).reshape(_EMB_D, 12 * 128)
    wlo = wcat[:_F]
    whi = jnp.pad(wcat[_F:], ((0, 2 * _F - _EMB_D), (0, 0)))

    emb_hi = _tc_split_emb(emb)
    e_lo, e_hi = _sc_embed(emb, emb_hi, ids)
    x_lsm = _tc_conv(e_lo.reshape(_B, _L, _F), e_hi.reshape(_B, _L, _F),
                     wlo, whi, b3.reshape(1, 128), b4.reshape(1, 128),
                     b5.reshape(1, 128))

    fw1 = _tc_matmul(features, Wg1, 1000)
    s1 = _sc_segsum(fw1, src_p, dst_p, zeros)
    h = _tc_relu_combine(s1, bg1.reshape(1, _F), 1000)

    s2 = _sc_segsum(h, src_p, dst_p, zeros)
    lbl_lsm = _tc_final(s2, Wg2, bg2.reshape(1, _C), 400)

    return jnp.concatenate([x_lsm, lbl_lsm], axis=0)


# speedup vs baseline: 1.0136x; 1.0136x over previous
import functools

import jax
import jax.numpy as jnp
from jax import lax
from jax.experimental import pallas as pl
from jax.experimental.pallas import tpu as pltpu
from jax.experimental.pallas import tpu_sc as plsc

_VOCAB = 100000
_EMB_D = 200
_B, _L = 16, 512
_N = 10000
_E = 320000
_F = 128
_C = 384

_NC, _NS = 2, 16
_NW = _NC * _NS
_CHUNK = 128
_K_SEG = -(-_E // (_NW * _CHUNK))
_E_PAD = _NW * _K_SEG * _CHUNK
_ACC_ROWS = 10240
_ZROWS = _ACC_ROWS // _NS

_IDS = _B * _L
_K_EMB = _IDS // (_NW * _CHUNK)


@functools.partial(
    pl.kernel,
    out_type=[jax.ShapeDtypeStruct((_IDS, _F), jnp.float32),
              jax.ShapeDtypeStruct((_IDS, _F), jnp.float32)],
    mesh=plsc.VectorSubcoreMesh(core_axis_name="c", subcore_axis_name="s"),
    scratch_types=[
        pltpu.VMEM((_K_EMB, _CHUNK), jnp.int32),
        pltpu.VMEM((_CHUNK, _F), jnp.float32),
        pltpu.VMEM((_CHUNK, _F), jnp.float32),
        pltpu.SemaphoreType.DMA,
    ],
)
def _sc_embed(emb_hbm, hi_hbm, ids_hbm, lo_out, hi_out, ids_v, rlo, rhi, sem):
    wid = lax.axis_index("c") * _NS + lax.axis_index("s")
    pltpu.sync_copy(ids_hbm.at[wid], ids_v)
    for j in range(_K_EMB):
        base = wid * (_K_EMB * _CHUNK) + j * _CHUNK
        pltpu.async_copy(
            emb_hbm.at[ids_v.at[j], pl.ds(0, _F)], rlo, sem).wait()
        pltpu.sync_copy(rlo, lo_out.at[pl.ds(base, _CHUNK)])
        pltpu.async_copy(hi_hbm.at[ids_v.at[j]], rhi, sem).wait()
        pltpu.sync_copy(rhi, hi_out.at[pl.ds(base, _CHUNK)])


@functools.partial(
    pl.kernel,
    out_type=jax.ShapeDtypeStruct((_NC, _ACC_ROWS, _F), jnp.float32),
    mesh=plsc.VectorSubcoreMesh(core_axis_name="c", subcore_axis_name="s"),
    scratch_types=[
        pltpu.VMEM((_K0, _CHUNK), jnp.int32),
        pltpu.VMEM((_K0, _CHUNK), jnp.int32),
        pltpu.VMEM((_CHUNK, _F), jnp.float32),
        pltpu.VMEM_SHARED((_ACC_ROWS, _F), jnp.float32),
        pltpu.SemaphoreType.DMA,
    ],
)
def _sc_segsum(x_hbm, src_hbm, dst_hbm, zeros_hbm, out_hbm,
               src_v, dst_v, rows_v, acc_sh, sem):
    cid = lax.axis_index("c")
    sid = lax.axis_index("s")
    base = (1 - cid) * sid * _K0 + cid * (_NS * _K0 + sid * _K1)
    pltpu.sync_copy(src_hbm.at[pl.ds(base, _K0)], src_v)
    pltpu.sync_copy(dst_hbm.at[pl.ds(base, _K0)], dst_v)
    pltpu.sync_copy(zeros_hbm, acc_sh.at[pl.ds(sid * _ZROWS, _ZROWS)])
    plsc.subcore_barrier()

    def body(j, carry):
        pltpu.async_copy(x_hbm.at[src_v.at[j]], rows_v, sem).wait()
        pltpu.sync_copy(rows_v, acc_sh.at[dst_v.at[j]], add=True)
        return carry

    @pl.when(cid == 0)
    def _():
        lax.fori_loop(0, _K0, body, 0)

    @pl.when(cid == 1)
    def _():
        lax.fori_loop(0, _K1, body, 0)

    plsc.subcore_barrier()
    pltpu.sync_copy(acc_sh.at[pl.ds(sid * _ZROWS, _ZROWS)],
                    out_hbm.at[cid, pl.ds(sid * _ZROWS, _ZROWS)])


def _tc_conv_body(elo_ref, ehi_ref, wlo_ref, whi_ref, b3_ref, b4_ref, b5_ref,
                  out_ref):
    y = (jnp.dot(elo_ref[0], wlo_ref[...], preferred_element_type=jnp.float32)
         + jnp.dot(ehi_ref[0], whi_ref[...], preferred_element_type=jnp.float32))
    c3 = y[0:510, 0:128] + y[1:511, 128:256] + y[2:512, 256:384]
    m3 = jnp.max(c3, axis=0, keepdims=True) + b3_ref[...]
    c4 = (y[0:509, 384:512] + y[1:510, 512:640]
          + y[2:511, 640:768] + y[3:512, 768:896])
    m4 = jnp.max(c4, axis=0, keepdims=True) + b4_ref[...]
    c5 = (y[0:508, 896:1024] + y[1:509, 1024:1152] + y[2:510, 1152:1280]
          + y[3:511, 1280:1408] + y[4:512, 1408:1536])
    m5 = jnp.max(c5, axis=0, keepdims=True) + b5_ref[...]
    row = jnp.maximum(jnp.concatenate([m3, m4, m5], axis=1), 0.0)
    mx = jnp.max(row, axis=1, keepdims=True)
    ls = row - mx - jnp.log(jnp.sum(jnp.exp(row - mx), axis=1, keepdims=True))
    out_ref[pl.ds(pl.program_id(0), 1), :] = ls


def _tc_conv(e_lo, e_hi, wlo, whi, b3, b4, b5):
    return pl.pallas_call(
        _tc_conv_body,
        grid=(_B,),
        in_specs=[
            pl.BlockSpec((1, _L, _F), lambda b: (b, 0, 0)),
            pl.BlockSpec((1, _L, _F), lambda b: (b, 0, 0)),
            pl.BlockSpec((_F, 1536), lambda b: (0, 0)),
            pl.BlockSpec((_F, 1536), lambda b: (0, 0)),
            pl.BlockSpec((1, 128), lambda b: (0, 0)),
            pl.BlockSpec((1, 128), lambda b: (0, 0)),
            pl.BlockSpec((1, 128), lambda b: (0, 0)),
        ],
        out_specs=pl.BlockSpec((_B, _C), lambda b: (0, 0)),
        out_shape=jax.ShapeDtypeStruct((_B, _C), jnp.float32),
    )(e_lo, e_hi, wlo, whi, b3, b4, b5)


def _tc_split_body(x_ref, hi_ref):
    hi_ref[...] = jnp.concatenate(
        [x_ref[:, _F:_EMB_D],
         jnp.zeros((x_ref.shape[0], 2 * _F - _EMB_D), jnp.float32)], axis=1)


def _tc_split_emb(emb, rows_blk=2000):
    return pl.pallas_call(
        _tc_split_body,
        grid=(_VOCAB // rows_blk,),
        in_specs=[pl.BlockSpec((rows_blk, _EMB_D), lambda i: (i, 0))],
        out_specs=pl.BlockSpec((rows_blk, _F), lambda i: (i, 0)),
        out_shape=jax.ShapeDtypeStruct((_VOCAB, _F), jnp.float32),
    )(emb)


def _tc_matmul_body(x_ref, w_ref, o_ref):
    o_ref[...] = jnp.dot(x_ref[...], w_ref[...],
                         preferred_element_type=jnp.float32)


def _tc_matmul(x, w, rows_blk):
    n, d = x.shape
    return pl.pallas_call(
        _tc_matmul_body,
        grid=(n // rows_blk,),
        in_specs=[
            pl.BlockSpec((rows_blk, d), lambda i: (i, 0)),
            pl.BlockSpec(w.shape, lambda i: (0, 0)),
        ],
        out_specs=pl.BlockSpec((rows_blk, w.shape[1]), lambda i: (i, 0)),
        out_shape=jax.ShapeDtypeStruct((n, w.shape[1]), jnp.float32),
    )(x, w)


def _tc_relu_body(a_ref, b_ref, bias_ref, o_ref):
    o_ref[...] = jnp.maximum(a_ref[0] + b_ref[0] + bias_ref[...], 0.0)


def _tc_relu_combine(p, bias, rows_blk, n=_N):
    d = p.shape[2]
    return pl.pallas_call(
        _tc_relu_body,
        grid=(n // rows_blk,),
        in_specs=[
            pl.BlockSpec((1, rows_blk, d), lambda i: (0, i, 0)),
            pl.BlockSpec((1, rows_blk, d), lambda i: (1, i, 0)),
            pl.BlockSpec((1, d), lambda i: (0, 0)),
        ],
        out_specs=pl.BlockSpec((rows_blk, d), lambda i: (i, 0)),
        out_shape=jax.ShapeDtypeStruct((n, d), jnp.float32),
    )(p, p, bias)


def _tc_final_body(a_ref, b_ref, w_ref, bias_ref, o_ref):
    s = a_ref[0] + b_ref[0]
    y = jnp.dot(s, w_ref[...], preferred_element_type=jnp.float32) + bias_ref[...]
    mx = jnp.max(y, axis=1, keepdims=True)
    o_ref[...] = y - mx - jnp.log(
        jnp.sum(jnp.exp(y - mx), axis=1, keepdims=True))


def _tc_final(p, w, bias, rows_blk, n=_N):
    d = p.shape[2]
    return pl.pallas_call(
        _tc_final_body,
        grid=(n // rows_blk,),
        in_specs=[
            pl.BlockSpec((1, rows_blk, d), lambda i: (0, i, 0)),
            pl.BlockSpec((1, rows_blk, d), lambda i: (1, i, 0)),
            pl.BlockSpec(w.shape, lambda i: (0, 0)),
            pl.BlockSpec((1, w.shape[1]), lambda i: (0, 0)),
        ],
        out_specs=pl.BlockSpec((rows_blk, w.shape[1]), lambda i: (i, 0)),
        out_shape=jax.ShapeDtypeStruct((n, w.shape[1]), jnp.float32),
    )(p, p, w, bias)


def kernel(input_seq, edge_index, features, emb, W3, b3, W4, b4, W5, b5,
           Wg1, bg1, Wg2, bg2):
    f32 = jnp.float32
    ids = input_seq.astype(jnp.int32).reshape(_NW, _K_EMB, _CHUNK)
    src = edge_index[0].astype(jnp.int32)
    dst = edge_index[1].astype(jnp.int32)
    src_p = jnp.pad(src, (0, _E_PAD - _E)).reshape(_NW, _K_SEG, _CHUNK)
    dst_p = jnp.pad(dst, (0, _E_PAD - _E),
                    constant_values=_N).reshape(_NW, _K_SEG, _CHUNK)
    zeros = jnp.zeros((_ZROWS, _F), f32)
    w3t = jnp.transpose(W3[:, 0], (1, 2, 0))
    w4t = jnp.transpose(W4[:, 0], (1, 2, 0))
    w5t = jnp.transpose(W5[:, 0], (1, 2, 0))
    wcat = jnp.concatenate([w3t, w4t, w5t], axis=0)
    wcat = jnp.transpose(wcat, (1, 0, 2)).reshape(_EMB_D, 12 * 128)
    wlo = wcat[:_F]
    whi = jnp.pad(wcat[_F:], ((0, 2 * _F - _EMB_D), (0, 0)))

    emb_hi = _tc_split_emb(emb)
    e_lo, e_hi = _sc_embed(emb, emb_hi, ids)
    x_lsm = _tc_conv(e_lo.reshape(_B, _L, _F), e_hi.reshape(_B, _L, _F),
                     wlo, whi, b3.reshape(1, 128), b4.reshape(1, 128),
                     b5.reshape(1, 128))

    fw1 = _tc_matmul(features, Wg1, 1000)
    s1 = _sc_segsum(fw1, src_p, dst_p, zeros)
    h = _tc_relu_combine(s1, bg1.reshape(1, _F), 1000)

    s2 = _sc_segsum(h, src_p, dst_p, zeros)
    lbl_lsm = _tc_final(s2, Wg2, bg2.reshape(1, _C), 400)

    return jnp.concatenate([x_lsm, lbl_lsm], axis=0)
